# Initial kernel scaffold; baseline (speedup 1.0000x reference)
import jax
import jax.numpy as jnp
from jax.experimental import pallas as pl


def kernel(pos_origin, pos, x, batch, W, gamma, beta):
    raise NotImplementedError("write your pallas kernel here")


# speedup vs baseline: 16.5224x; 16.5224x over previous
import functools
import math

import jax
import jax.numpy as jnp
from jax import lax
from jax.experimental import pallas as pl
from jax.experimental.pallas import tpu as pltpu
from jax.experimental.pallas import tpu_sc as plsc

N = 10000
IN_DIM = 128
OUT_DIM = 256
N_SAMPLES = int(math.ceil(N * 0.25))
S = N_SAMPLES - 1
R2 = 0.1 * 0.1

NPAD = 10240
ROWS8 = 8
COLS = NPAD // ROWS8

NC = 2
NS = 16
NW = NC * NS
L = 16
NQ = 80
SQ = NW * NQ
KCAP = 128
CHUNK = 32
NCH_PTS = NPAD // L


def _feats_body(x_ref, wt_ref, g_ref, b_ref, o_ref):
    h = jnp.dot(x_ref[...], wt_ref[...], preferred_element_type=jnp.float32)
    mu = jnp.mean(h, axis=-1, keepdims=True)
    var = jnp.mean((h - mu) ** 2, axis=-1, keepdims=True)
    y = (h - mu) / jnp.sqrt(var + 1e-5) * g_ref[...] + b_ref[...]
    o_ref[...] = jnp.maximum(y, 0.0)


def _feats(x_pad, wt, gamma2, beta2):
    blk = 512
    grid = NPAD // blk
    return pl.pallas_call(
        _feats_body,
        grid=(grid,),
        in_specs=[
            pl.BlockSpec((blk, IN_DIM), lambda i: (i, 0)),
            pl.BlockSpec((IN_DIM, OUT_DIM), lambda i: (0, 0)),
            pl.BlockSpec((1, OUT_DIM), lambda i: (0, 0)),
            pl.BlockSpec((1, OUT_DIM), lambda i: (0, 0)),
        ],
        out_specs=pl.BlockSpec((blk, OUT_DIM), lambda i: (i, 0)),
        out_shape=jax.ShapeDtypeStruct((NPAD, OUT_DIM), jnp.float32),
    )(x_pad, wt, gamma2, beta2)


def _fps_body(pxv, pyv, pzv, pxs, pys, pzs,
              idx_out, qx_out, qy_out, qz_out, d_ref):
    iota_r = lax.broadcasted_iota(jnp.int32, (ROWS8, COLS), 0)
    iota_c = lax.broadcasted_iota(jnp.int32, (ROWS8, COLS), 1)
    flat = iota_r * COLS + iota_c
    valid = flat < N

    x0 = pxs[0]
    y0 = pys[0]
    z0 = pzs[0]
    dx = pxv[...] - x0
    dy = pyv[...] - y0
    dz = pzv[...] - z0
    d0 = (dx * dx + dz * dz) + dy * dy
    d_ref[...] = jnp.where(valid, d0, -1.0)
    idx_out[0] = 0
    qx_out[0] = x0
    qy_out[0] = y0
    qz_out[0] = z0

    def body(i, carry):
        d = d_ref[...]
        m = jnp.max(d)
        nxt = jnp.min(jnp.where(d == m, flat, jnp.int32(2 ** 30)))
        idx_out[i] = nxt
        sx = pxs[nxt]
        sy = pys[nxt]
        sz = pzs[nxt]
        qx_out[i] = sx
        qy_out[i] = sy
        qz_out[i] = sz
        ddx = pxv[...] - sx
        ddy = pyv[...] - sy
        ddz = pzv[...] - sz
        dn = (ddx * ddx + ddz * ddz) + ddy * ddy
        d_ref[...] = jnp.minimum(d, dn)
        return carry

    lax.fori_loop(1, N_SAMPLES, body, 0)


def _fps(pox, poy, poz):
    f32 = jnp.float32
    return pl.pallas_call(
        _fps_body,
        in_specs=[
            pl.BlockSpec((ROWS8, COLS), lambda: (0, 0)),
            pl.BlockSpec((ROWS8, COLS), lambda: (0, 0)),
            pl.BlockSpec((ROWS8, COLS), lambda: (0, 0)),
            pl.BlockSpec(memory_space=pltpu.SMEM),
            pl.BlockSpec(memory_space=pltpu.SMEM),
            pl.BlockSpec(memory_space=pltpu.SMEM),
        ],
        out_specs=[
            pl.BlockSpec(memory_space=pltpu.SMEM),
            pl.BlockSpec(memory_space=pltpu.SMEM),
            pl.BlockSpec(memory_space=pltpu.SMEM),
            pl.BlockSpec(memory_space=pltpu.SMEM),
        ],
        out_shape=[
            jax.ShapeDtypeStruct((N_SAMPLES,), jnp.int32),
            jax.ShapeDtypeStruct((N_SAMPLES,), f32),
            jax.ShapeDtypeStruct((N_SAMPLES,), f32),
            jax.ShapeDtypeStruct((N_SAMPLES,), f32),
        ],
        scratch_shapes=[pltpu.VMEM((ROWS8, COLS), jnp.float32)],
    )(pox.reshape(ROWS8, COLS), poy.reshape(ROWS8, COLS),
      poz.reshape(ROWS8, COLS), pox, poy, poz)


def _mask_body(qm_ref, pt_ref, qx_ref, qy_ref, qz_ref,
               px_ref, py_ref, pz_ref, o_ref):
    qx = qx_ref[...]
    qy = qy_ref[...]
    qz = qz_ref[...]
    q2 = (qx * qx + qz * qz) + qy * qy
    px = px_ref[...]
    py = py_ref[...]
    pz = pz_ref[...]
    p2 = (px * px + pz * pz) + py * py
    qp = jnp.dot(qm_ref[...].astype(jnp.bfloat16),
                 pt_ref[...].astype(jnp.bfloat16),
                 preferred_element_type=jnp.float32)
    dist2 = (q2 + p2) - 2.0 * qp
    o_ref[...] = jnp.where(dist2 <= R2, jnp.maximum(dist2, 0.0), 1e9)


def _mask(qmat, pmat_t, qx, qy, qz, pox, poy, poz):
    blk = 256
    grid = SQ // blk
    return pl.pallas_call(
        _mask_body,
        grid=(grid,),
        in_specs=[
            pl.BlockSpec((blk, 8), lambda i: (i, 0)),
            pl.BlockSpec((8, NPAD), lambda i: (0, 0)),
            pl.BlockSpec((blk, 1), lambda i: (i, 0)),
            pl.BlockSpec((blk, 1), lambda i: (i, 0)),
            pl.BlockSpec((blk, 1), lambda i: (i, 0)),
            pl.BlockSpec((1, NPAD), lambda i: (0, 0)),
            pl.BlockSpec((1, NPAD), lambda i: (0, 0)),
            pl.BlockSpec((1, NPAD), lambda i: (0, 0)),
        ],
        out_specs=pl.BlockSpec((blk, NPAD), lambda i: (i, 0)),
        out_shape=jax.ShapeDtypeStruct((SQ, NPAD), jnp.float32),
    )(qmat, pmat_t, qx, qy, qz, pox, poy, poz)


def _lane_extract_i32(vec, lane):
    lanes = lax.iota(jnp.int32, L)
    return lax.reduce_max(jnp.where(lanes == lane, vec, jnp.int32(-2 ** 31 + 1)),
                          axes=(0,))


MAXNB = 64
KEY_HI = 1008981771


def _sc_body(qidx_h, mask_h, p2x_h, p2y_h, p2z_h, bat_h, feats_h,
             xout_h, q2x_h, q2y_h, q2z_h, qb_h,
             p2x_v, p2y_v, p2z_v, bat_v,
             qidx_v, q2_v, qb_v,
             mask_v, nbr_v, nbrd_v, sel_v, rows_v, xout_v, sem):
    wid = lax.axis_index("s") * NC + lax.axis_index("c")
    qbase = wid * NQ

    pltpu.sync_copy(p2x_h, p2x_v)
    pltpu.sync_copy(p2y_h, p2y_v)
    pltpu.sync_copy(p2z_h, p2z_v)
    pltpu.sync_copy(bat_h, bat_v)
    pltpu.sync_copy(qidx_h.at[pl.ds(qbase, NQ)], qidx_v)

    def gather_chunk(c, carry):
        sl = pl.ds(c * L, L)
        qi = qidx_v[sl]
        q2_v[0, sl] = plsc.load_gather(p2x_v, [qi])
        q2_v[1, sl] = plsc.load_gather(p2y_v, [qi])
        q2_v[2, sl] = plsc.load_gather(p2z_v, [qi])
        qb_v[sl] = plsc.load_gather(bat_v, [qi])
        return carry

    lax.fori_loop(0, NQ // L, gather_chunk, 0)

    lanes = lax.iota(jnp.int32, L)

    def per_query(qi, carry):
        qiv = qidx_v[pl.ds((qi // L) * L, L)]
        self_idx = _lane_extract_i32(qiv, qi % L)

        pltpu.sync_copy(mask_h.at[qbase + qi], mask_v)

        def prefill(k, c):
            nbr_v[pl.ds(k * L, L)] = jnp.broadcast_to(self_idx, (L,))
            nbrd_v[pl.ds(k * L, L)] = jnp.full((L,), 1e9, jnp.float32)
            return c
        lax.fori_loop(0, KCAP // L, prefill, 0)

        def scan_pts(c, cnt):
            m = mask_v[pl.ds(c * L, L)]
            msk = m <= R2
            incl = plsc.cumsum(jnp.where(msk, jnp.int32(1), jnp.int32(0)))
            posn = cnt + incl - 1
            posn = jnp.minimum(posn, jnp.int32(KCAP - 1))
            plsc.store_scatter(nbr_v, [posn], c * L + lanes, mask=msk)
            plsc.store_scatter(nbrd_v, [posn], m, mask=msk)
            return cnt + plsc.all_reduce_population_count(msk)

        cntv = lax.fori_loop(0, NCH_PTS, scan_pts,
                             jnp.zeros((L,), jnp.int32))
        cnt = _lane_extract_i32(cntv, 0)

        acc0 = tuple(jnp.zeros((L,), jnp.float32)
                     for _ in range(OUT_DIM // L))

        def gather_max(src_ref, nch, acc_init):
            def per_chunk(ch, acc):
                pltpu.async_copy(
                    feats_h.at[src_ref.at[pl.ds(ch * CHUNK, CHUNK)]],
                    rows_v, sem).wait()

                def per_row(r, acc2):
                    return tuple(
                        jnp.maximum(acc2[k], rows_v[r, pl.ds(k * L, L)])
                        for k in range(OUT_DIM // L))

                return lax.fori_loop(0, CHUNK, per_row, acc)

            return lax.fori_loop(0, nch, per_chunk, acc_init)

        def count_le(tvec):
            def cchunk(k, cv):
                d2k = nbrd_v[pl.ds(k * L, L)]
                return cv + plsc.all_reduce_population_count(d2k <= tvec)
            return lax.fori_loop(0, KCAP // L, cchunk,
                                 jnp.zeros((L,), jnp.int32))

        def sparse_path():
            nch = (cnt + (CHUNK - 1)) // CHUNK
            return gather_max(nbr_v, nch, acc0)

        def dense_path():
            def bis(i, lohi):
                lo, hi = lohi
                mid = (lo + hi) // 2
                tvec = plsc.bitcast(jnp.broadcast_to(mid, (L,)), jnp.float32)
                c = _lane_extract_i32(count_le(tvec), 0)
                big = c >= MAXNB
                return (jnp.where(big, lo, mid + 1), jnp.where(big, mid, hi))

            _, kstar = lax.fori_loop(
                0, 30, bis, (jnp.int32(0), jnp.int32(KEY_HI)))
            tvec = plsc.bitcast(jnp.broadcast_to(kstar, (L,)), jnp.float32)

            def clt(k, cv):
                d2k = nbrd_v[pl.ds(k * L, L)]
                return cv + plsc.all_reduce_population_count(d2k < tvec)
            c1 = _lane_extract_i32(
                lax.fori_loop(0, KCAP // L, clt, jnp.zeros((L,), jnp.int32)),
                0)
            take = MAXNB - c1

            def self_fill(k, c):
                sel_v[pl.ds(k * L, L)] = jnp.broadcast_to(self_idx, (L,))
                return c
            lax.fori_loop(0, MAXNB // L, self_fill, 0)

            def fchunk(k, cv):
                selcnt, tiecnt = cv
                d2k = nbrd_v[pl.ds(k * L, L)]
                idxk = nbr_v[pl.ds(k * L, L)]
                lt = d2k < tvec
                eq = d2k == tvec
                tie_incl = plsc.cumsum(
                    jnp.where(eq, jnp.int32(1), jnp.int32(0)))
                tiepos = tiecnt + tie_incl - 1
                keep = lt | (eq & (tiepos < take))
                incl = plsc.cumsum(
                    jnp.where(keep, jnp.int32(1), jnp.int32(0)))
                posn = jnp.minimum(selcnt + incl - 1, jnp.int32(MAXNB - 1))
                plsc.store_scatter(sel_v, [posn], idxk, mask=keep)
                return (selcnt + plsc.all_reduce_population_count(keep),
                        tiecnt + plsc.all_reduce_population_count(eq))

            lax.fori_loop(0, KCAP // L, fchunk,
                          (jnp.zeros((L,), jnp.int32),
                           jnp.zeros((L,), jnp.int32)))
            return gather_max(sel_v, MAXNB // CHUNK, acc0)

        acc = lax.cond(cnt > MAXNB, dense_path, sparse_path)
        for k in range(OUT_DIM // L):
            xout_v[qi, pl.ds(k * L, L)] = acc[k]
        return carry

    lax.fori_loop(0, NQ, per_query, 0)

    pltpu.sync_copy(xout_v, xout_h.at[pl.ds(qbase, NQ)])
    pltpu.sync_copy(q2_v.at[0], q2x_h.at[pl.ds(qbase, NQ)])
    pltpu.sync_copy(q2_v.at[1], q2y_h.at[pl.ds(qbase, NQ)])
    pltpu.sync_copy(q2_v.at[2], q2z_h.at[pl.ds(qbase, NQ)])
    pltpu.sync_copy(qb_v, qb_h.at[pl.ds(qbase, NQ)])


def _sc_pool(qidx, mask, p2x, p2y, p2z, bat, feats):
    mesh = plsc.VectorSubcoreMesh(core_axis_name="c", subcore_axis_name="s",
                                  num_cores=NC, num_subcores=NS)
    f32 = jnp.float32
    kern = pl.kernel(
        _sc_body,
        out_type=[
            jax.ShapeDtypeStruct((SQ, OUT_DIM), f32),
            jax.ShapeDtypeStruct((SQ,), f32),
            jax.ShapeDtypeStruct((SQ,), f32),
            jax.ShapeDtypeStruct((SQ,), f32),
            jax.ShapeDtypeStruct((SQ,), jnp.int32),
        ],
        mesh=mesh,
        compiler_params=pltpu.CompilerParams(needs_layout_passes=False),
        scratch_types=[
            pltpu.VMEM((NPAD,), f32), pltpu.VMEM((NPAD,), f32),
            pltpu.VMEM((NPAD,), f32),
            pltpu.VMEM((NPAD,), jnp.int32),
            pltpu.VMEM((NQ,), jnp.int32),
            pltpu.VMEM((3, NQ), f32),
            pltpu.VMEM((NQ,), jnp.int32),
            pltpu.VMEM((NPAD,), f32),
            pltpu.VMEM((KCAP,), jnp.int32),
            pltpu.VMEM((KCAP,), f32),
            pltpu.VMEM((MAXNB,), jnp.int32),
            pltpu.VMEM((CHUNK, OUT_DIM), f32),
            pltpu.VMEM((NQ, OUT_DIM), f32),
            pltpu.SemaphoreType.DMA,
        ],
    )
    return kern(qidx, mask, p2x, p2y, p2z, bat, feats)


def kernel(pos_origin, pos, x, batch, W, gamma, beta):
    f32 = jnp.float32
    pad = NPAD - N
    pox = jnp.pad(pos_origin[:, 0], (0, pad), constant_values=1e9)
    poy = jnp.pad(pos_origin[:, 1], (0, pad), constant_values=1e9)
    poz = jnp.pad(pos_origin[:, 2], (0, pad), constant_values=1e9)
    p2x = jnp.pad(pos[:, 0], (0, pad))
    p2y = jnp.pad(pos[:, 1], (0, pad))
    p2z = jnp.pad(pos[:, 2], (0, pad))
    bat = jnp.pad(batch, (0, pad))
    x_pad = jnp.pad(x, ((0, pad), (0, 0)))

    feats = _feats(x_pad, W.T, gamma.reshape(1, OUT_DIM),
                   beta.reshape(1, OUT_DIM))
    samples, qox, qoy, qoz = _fps(pox, poy, poz)
    qidx = jnp.pad(samples[1:], (0, SQ - S))

    far = jnp.full((SQ - S,), 2e9, f32)
    qx = jnp.concatenate([qox[1:], far])
    qy = jnp.concatenate([qoy[1:], far])
    qz = jnp.concatenate([qoz[1:], far])
    zeros_q = jnp.zeros((SQ, 5), f32)
    qmat = jnp.concatenate([qx[:, None], qy[:, None], qz[:, None], zeros_q],
                           axis=1)
    zeros_p = jnp.zeros((5, NPAD), f32)
    pmat_t = jnp.concatenate([pox[None, :], poy[None, :], poz[None, :],
                              zeros_p], axis=0)

    mask = _mask(qmat, pmat_t, qx[:, None], qy[:, None], qz[:, None],
                 pox[None, :], poy[None, :], poz[None, :])

    (xout, q2x, q2y, q2z, qb) = _sc_pool(qidx, mask, p2x, p2y, p2z, bat, feats)

    pos_origin_out = jnp.stack([qox[1:], qoy[1:], qoz[1:]], axis=1)
    pos_out = jnp.stack([q2x[:S], q2y[:S], q2z[:S]], axis=1)
    return (pos_origin_out, pos_out, xout[:S], qb[:S])
